# baseline (speedup 1.0000x reference)
import functools

import jax
import jax.numpy as jnp
from jax import lax
from jax.experimental import pallas as pl
from jax.experimental.pallas import tpu as pltpu
from jax.experimental.pallas import tpu_sc as plsc

_R, _C = 200, 16384
_NW = 32
_WC = _C // _NW
_CHR = 40
_NCH = _R // _CHR
_L = 16

_mesh = plsc.VectorSubcoreMesh(core_axis_name="c", subcore_axis_name="s")


@functools.partial(
    pl.kernel,
    mesh=_mesh,
    out_type=jax.ShapeDtypeStruct((_R, _C), jnp.float32),
    scratch_types=[
        pltpu.VMEM((_L,), jnp.float32),
        pltpu.VMEM((_CHR, _WC), jnp.int32),
        pltpu.VMEM((_CHR, _WC), jnp.int32),
        pltpu.VMEM((_CHR, _WC), jnp.int32),
        pltpu.VMEM((_CHR, _WC), jnp.float32),
        pltpu.VMEM((_CHR, _WC), jnp.float32),
        pltpu.VMEM((_CHR, _WC), jnp.float32),
        pltpu.SemaphoreType.DMA,
        pltpu.SemaphoreType.DMA,
        pltpu.SemaphoreType.DMA,
        pltpu.SemaphoreType.DMA,
        pltpu.SemaphoreType.DMA,
        pltpu.SemaphoreType.DMA,
    ],
    compiler_params=pltpu.CompilerParams(use_tc_tiling_on_sc=True),
)
def _select_kernel(in_hbm, a_hbm, out_hbm, a_v, in0, in1, in2, out0, out1,
                   out2, si0, si1, si2, so0, so1, so2):
    wid = lax.axis_index("s") * 2 + lax.axis_index("c")
    base = wid * _WC

    pltpu.sync_copy(a_hbm.at[pl.ds(0, _L)], a_v)
    av = a_v[...]
    a0 = jnp.broadcast_to(av[0], (_L,))
    a1 = jnp.broadcast_to(av[1], (_L,))

    in_bufs, out_bufs = (in0, in1, in2), (out0, out1, out2)
    in_sems, out_sems = (si0, si1, si2), (so0, so1, so2)

    def start_in(ch):
        return pltpu.async_copy(
            in_hbm.at[pl.ds(ch * _CHR, _CHR), pl.ds(base, _WC)],
            in_bufs[ch % 3], in_sems[ch % 3])

    descs_in = [None] * _NCH
    descs_out = [None] * _NCH
    descs_in[0] = start_in(0)
    descs_in[1] = start_in(1)
    for ch in range(_NCH):
        b = ch % 3
        if ch + 2 < _NCH:
            descs_in[ch + 2] = start_in(ch + 2)
        descs_in[ch].wait()
        if ch >= 3:
            descs_out[ch - 3].wait()
        in_v, out_v = in_bufs[b], out_bufs[b]

        @plsc.parallel_loop(0, _CHR, step=1, unroll=1)
        def body(r):
            for c in range(0, _WC, _L):
                x = in_v[r, pl.ds(c, _L)]
                out_v[r, pl.ds(c, _L)] = jnp.where(x > 0, a1, a0)

        descs_out[ch] = pltpu.async_copy(
            out_v, out_hbm.at[pl.ds(ch * _CHR, _CHR), pl.ds(base, _WC)],
            out_sems[b])
    descs_out[_NCH - 3].wait()
    descs_out[_NCH - 2].wait()
    descs_out[_NCH - 1].wait()


def kernel(input_tensor, A):
    return _select_kernel(input_tensor.T, A).T
